# Initial kernel scaffold; baseline (speedup 1.0000x reference)
import jax
import jax.numpy as jnp
from jax.experimental import pallas as pl


def kernel(x, y, table):
    raise NotImplementedError("write your pallas kernel here")


# speedup vs baseline: 3.5194x; 3.5194x over previous
import functools

import jax
import jax.numpy as jnp
from jax import lax
from jax.experimental import pallas as pl
from jax.experimental.pallas import tpu as pltpu
from jax.experimental.pallas import tpu_sc as plsc

_B, _T, _C = 1024, 50, 1024
_BT = _B * _T
_NC, _NS, _L = 2, 16, 16
_NW = _NC * _NS
_BPW = _BT // _NW
_CH = 32
_NCH = _BPW // _CH


def _lse_body(table_ref, lse_ref):
    t = table_ref[...]
    m = jnp.max(t, axis=1, keepdims=True)
    s = jnp.sum(jnp.exp(t - m), axis=1, keepdims=True)
    lse_ref[...] = (jnp.log(s) + m)[:, 0]


_lse_call = pl.pallas_call(
    _lse_body,
    out_shape=jax.ShapeDtypeStruct((_C,), jnp.float32),
)


def _sc_body(x_hbm, y_hbm, table_hbm, lse_hbm, out_hbm, part_hbm,
             idx_v, y_v, lse_v, rows_v, acc_v, sem):
    wid = lax.axis_index("s") * _NC + lax.axis_index("c")
    base = wid * _BPW
    pltpu.sync_copy(lse_hbm, lse_v)
    pltpu.sync_copy(x_hbm.at[pl.ds(base, _BPW)], idx_v)
    pltpu.sync_copy(y_hbm.at[pl.ds(base, _BPW)], y_v)
    acc_v[...] = jnp.zeros((_L,), jnp.float32)

    @pl.loop(0, _NCH)
    def _chunk(c):
        off = c * _CH
        pltpu.async_copy(table_hbm.at[idx_v.at[pl.ds(off, _CH)]], rows_v,
                         sem).wait()
        for g in range(_CH // _L):
            xv = idx_v[pl.ds(off + g * _L, _L)]
            yv = y_v[pl.ds(off + g * _L, _L)]
            rid = lax.iota(jnp.int32, _L) + g * _L
            lseg = plsc.load_gather(lse_v, [xv])
            val = plsc.load_gather(rows_v, [rid, yv])
            acc_v[...] = acc_v[...] + (lseg - val)
        pltpu.sync_copy(rows_v, out_hbm.at[pl.ds(base + off, _CH)])

    pltpu.sync_copy(acc_v, part_hbm.at[wid])


_sc_call = pl.kernel(
    _sc_body,
    out_type=[jax.ShapeDtypeStruct((_BT, _C), jnp.float32),
              jax.ShapeDtypeStruct((_NW, _L), jnp.float32)],
    mesh=plsc.VectorSubcoreMesh(core_axis_name="c", subcore_axis_name="s",
                                num_cores=_NC, num_subcores=_NS),
    compiler_params=pltpu.CompilerParams(needs_layout_passes=False),
    scratch_types=[
        pltpu.VMEM((_BPW,), jnp.int32),
        pltpu.VMEM((_BPW,), jnp.int32),
        pltpu.VMEM((_C,), jnp.float32),
        pltpu.VMEM((_CH, _C), jnp.float32),
        pltpu.VMEM((_L,), jnp.float32),
        pltpu.SemaphoreType.DMA,
    ],
)


def kernel(x, y, table):
    xf = x.reshape(_BT)
    yf = y.reshape(_BT)
    lse = _lse_call(table)
    logits2, part = _sc_call(xf, yf, table, lse)
    loss = jnp.sum(part) / _BT
    return (logits2, loss)


# speedup vs baseline: 4.1107x; 1.1680x over previous
import functools

import jax
import jax.numpy as jnp
from jax import lax
from jax.experimental import pallas as pl
from jax.experimental.pallas import tpu as pltpu
from jax.experimental.pallas import tpu_sc as plsc

_B, _T, _C = 1024, 50, 1024
_BT = _B * _T
_NC, _NS, _L = 2, 16, 16
_NW = _NC * _NS
_BPW = _BT // _NW
_CH = 16
_NCH = _BPW // _CH
_NBUF = 4


def _lse_body(table_ref, lse_ref):
    t = table_ref[...]
    m = jnp.max(t, axis=1, keepdims=True)
    s = jnp.sum(jnp.exp(t - m), axis=1, keepdims=True)
    lse_ref[...] = (jnp.log(s) + m)[:, 0]


_lse_call = pl.pallas_call(
    _lse_body,
    out_shape=jax.ShapeDtypeStruct((_C,), jnp.float32),
)


def _sc_body(x_hbm, y_hbm, table_hbm, lse_hbm, out_hbm, part_hbm,
             idx_v, y_v, lse_v, rows_v, acc_v, gsems, ssems):
    wid = lax.axis_index("s") * _NC + lax.axis_index("c")
    base = wid * _BPW
    pltpu.sync_copy(lse_hbm, lse_v)
    pltpu.sync_copy(x_hbm.at[pl.ds(base, _BPW)], idx_v)
    pltpu.sync_copy(y_hbm.at[pl.ds(base, _BPW)], y_v)
    acc_v[...] = jnp.zeros((_L,), jnp.float32)

    def gather(c, b):
        return pltpu.make_async_copy(
            table_hbm.at[idx_v.at[pl.ds(c * _CH, _CH)]], rows_v.at[b],
            gsems.at[b])

    def scatter(c, b):
        return pltpu.make_async_copy(
            rows_v.at[b], out_hbm.at[pl.ds(base + c * _CH, _CH)], ssems.at[b])

    for r in range(_NBUF - 1):
        gather(r, r).start()

    @pl.loop(0, _NCH, step=_NBUF)
    def _outer(c0):
        for b in range(_NBUF):
            c = c0 + b
            gather(c, b).wait()
            xv = idx_v[pl.ds(c * _CH, _L)]
            yv = y_v[pl.ds(c * _CH, _L)]
            rid = lax.iota(jnp.int32, _L)
            lseg = plsc.load_gather(lse_v, [xv])
            val = plsc.load_gather(rows_v.at[b], [rid, yv])
            acc_v[...] = acc_v[...] + (lseg - val)
            scatter(c, b).start()
            nb = (b + _NBUF - 1) % _NBUF

            @pl.when(c + _NBUF - 1 < _NCH)
            def _():
                @pl.when(c >= 1)
                def _():
                    scatter(c - 1, nb).wait()
                gather(c + _NBUF - 1, nb).start()

    for b in range(_NBUF):
        scatter(_NCH - _NBUF + b, b).wait()

    pltpu.sync_copy(acc_v, part_hbm.at[wid])


_sc_call = pl.kernel(
    _sc_body,
    out_type=[jax.ShapeDtypeStruct((_BT, _C), jnp.float32),
              jax.ShapeDtypeStruct((_NW, _L), jnp.float32)],
    mesh=plsc.VectorSubcoreMesh(core_axis_name="c", subcore_axis_name="s",
                                num_cores=_NC, num_subcores=_NS),
    compiler_params=pltpu.CompilerParams(needs_layout_passes=False),
    scratch_types=[
        pltpu.VMEM((_BPW,), jnp.int32),
        pltpu.VMEM((_BPW,), jnp.int32),
        pltpu.VMEM((_C,), jnp.float32),
        pltpu.VMEM((_NBUF, _CH, _C), jnp.float32),
        pltpu.VMEM((_L,), jnp.float32),
        pltpu.SemaphoreType.DMA((_NBUF,)),
        pltpu.SemaphoreType.DMA((_NBUF,)),
    ],
)


def kernel(x, y, table):
    xf = x.reshape(_BT)
    yf = y.reshape(_BT)
    lse = _lse_call(table)
    logits2, part = _sc_call(xf, yf, table, lse)
    loss = jnp.sum(part) / _BT
    return (logits2, loss)
